# Initial kernel scaffold; baseline (speedup 1.0000x reference)
import jax
import jax.numpy as jnp
from jax.experimental import pallas as pl


def kernel(target_table, context_table, u_pos, v_pos, v_neg):
    raise NotImplementedError("write your pallas kernel here")


# speedup vs baseline: 1.7140x; 1.7140x over previous
import functools

import jax
import jax.numpy as jnp
from jax import lax
from jax.experimental import pallas as pl
from jax.experimental.pallas import tpu as pltpu
from jax.experimental.pallas import tpu_sc as plsc

NC = 2
NS = 16
NW = NC * NS
LANES = 16
CH = 128


def _sc_body(D, BW, NCH, NNEG,
             tgt_hbm, ctx_hbm, ui_hbm, vi_hbm, vn_hbm,
             pos_hbm, neg_hbm,
             idx_u, idx_vp, idx_vn, u_rows, vp_rows, vs_rows,
             pos_part, neg_part, sem_g, sem_n):
    wid = lax.axis_index("s") * NC + lax.axis_index("c")
    base = wid * BW

    pltpu.sync_copy(ui_hbm.at[wid], idx_u)
    pltpu.sync_copy(vi_hbm.at[wid], idx_vp)
    pltpu.sync_copy(vn_hbm.at[0, wid], idx_vn.at[0])

    cps = []
    for j in range(NCH):
        dst = pl.ds(j * CH, CH)
        cps.append(pltpu.async_copy(tgt_hbm.at[idx_u.at[j]], u_rows.at[dst], sem_g))
        cps.append(pltpu.async_copy(ctx_hbm.at[idx_vp.at[j]], vp_rows.at[dst], sem_g))

    ncps = [pltpu.async_copy(ctx_hbm.at[idx_vn.at[0, j]],
                             vs_rows.at[pl.ds(j * CH, CH)], sem_n)
            for j in range(NCH)]
    for n in range(1, NNEG):
        slot = n % 2
        pltpu.sync_copy(vn_hbm.at[n, wid], idx_vn.at[slot])
        for c in ncps:
            c.wait()
        ncps = [pltpu.async_copy(ctx_hbm.at[idx_vn.at[slot, j]],
                                 vs_rows.at[pl.ds(j * CH, CH)], sem_n, add=True)
                for j in range(NCH)]

    for c in cps:
        c.wait()

    def pos_body(i, carry):
        pp = jnp.zeros((LANES,), jnp.float32)
        for j in range(D // LANES):
            sl = pl.ds(j * LANES, LANES)
            pp = pp + u_rows[i, sl] * vp_rows[i, sl]
        pos_part[i, :] = pp
        return carry

    lax.fori_loop(0, BW, pos_body, 0)

    for c in ncps:
        c.wait()

    def neg_body(i, carry):
        pn = jnp.zeros((LANES,), jnp.float32)
        for j in range(D // LANES):
            sl = pl.ds(j * LANES, LANES)
            pn = pn + u_rows[i, sl] * vs_rows[i, sl]
        neg_part[i, :] = pn
        return carry

    lax.fori_loop(0, BW, neg_body, 0)

    pltpu.sync_copy(pos_part, pos_hbm.at[pl.ds(base, BW)])
    pltpu.sync_copy(neg_part, neg_hbm.at[pl.ds(base, BW)])


def _sc_scores(tgt, ctx, ui, vi, vn, B, D, NNEG):
    BW = B // NW
    NCH = BW // CH
    mesh = plsc.VectorSubcoreMesh(core_axis_name="c", subcore_axis_name="s",
                                  num_cores=NC, num_subcores=NS)
    f32 = jnp.float32
    return pl.kernel(
        functools.partial(_sc_body, D, BW, NCH, NNEG),
        out_type=[jax.ShapeDtypeStruct((B, LANES), f32),
                  jax.ShapeDtypeStruct((B, LANES), f32)],
        mesh=mesh,
        compiler_params=pltpu.CompilerParams(use_tc_tiling_on_sc=False),
        scratch_types=[
            pltpu.VMEM((NCH, CH), jnp.int32),
            pltpu.VMEM((NCH, CH), jnp.int32),
            pltpu.VMEM((2, NCH, CH), jnp.int32),
            pltpu.VMEM((BW, D), f32),
            pltpu.VMEM((BW, D), f32),
            pltpu.VMEM((BW, D), f32),
            pltpu.VMEM((BW, LANES), f32),
            pltpu.VMEM((BW, LANES), f32),
            pltpu.SemaphoreType.DMA,
            pltpu.SemaphoreType.DMA,
        ],
    )(tgt, ctx, ui, vi, vn)


def _loss_body(p_ref, n_ref, o_ref):
    ps = jnp.sum(p_ref[...], axis=1)
    ns = jnp.sum(n_ref[...], axis=1)
    cost = jax.nn.log_sigmoid(ps) + jax.nn.log_sigmoid(-ns)
    o_ref[0, 0] = -jnp.sum(cost) / ps.shape[0]


def _loss_tc(pos_part, neg_part):
    return pl.pallas_call(
        _loss_body,
        out_shape=jax.ShapeDtypeStruct((1, 1), jnp.float32),
        in_specs=[pl.BlockSpec(memory_space=pltpu.VMEM),
                  pl.BlockSpec(memory_space=pltpu.VMEM)],
        out_specs=pl.BlockSpec(memory_space=pltpu.SMEM),
    )(pos_part, neg_part)


def kernel(target_table, context_table, u_pos, v_pos, v_neg):
    B = u_pos.shape[0]
    D = target_table.shape[1]
    NNEG = v_neg.shape[1]
    BW = B // NW
    NCH = BW // CH
    ui = u_pos.astype(jnp.int32).reshape(NW, NCH, CH)
    vi = v_pos.astype(jnp.int32).reshape(NW, NCH, CH)
    vn = v_neg.astype(jnp.int32).T.reshape(NNEG, NW, NCH, CH)
    pos_part, neg_part = _sc_scores(target_table, context_table, ui, vi, vn,
                                    B, D, NNEG)
    loss = _loss_tc(pos_part, neg_part)
    return loss[0, 0]


# speedup vs baseline: 2.2133x; 1.2913x over previous
import functools

import jax
import jax.numpy as jnp
from jax import lax
from jax.experimental import pallas as pl
from jax.experimental.pallas import tpu as pltpu
from jax.experimental.pallas import tpu_sc as plsc

NC = 2
NS = 16
NW = NC * NS
LANES = 16
CH = 8
TR = 8
OCT = 8
IW = 64


def _sc_body(D, BW, NCH, NNEG,
             tgt_hbm, ctx_hbm, idx_hbm,
             pos_hbm, neg_hbm,
             idx_v, u2d, vp2d, vn2d, part_p, part_n,
             sem_u0, sem_u1, sem_v0, sem_v1, sem_n0, sem_n1):
    wid = lax.axis_index("s") * NC + lax.axis_index("c")
    sem_u = [sem_u0, sem_u1]
    sem_v = [sem_v0, sem_v1]
    sem_n = [sem_n0, sem_n1]

    irows = BW * IW // CH // 128
    pltpu.sync_copy(idx_hbm.at[pl.ds(wid * irows, irows)], idx_v)

    def win(off):
        return idx_v[off // 128, pl.ds(off % 128, LANES)]

    def chunk_idx(c):
        o = c * IW
        return win(o), win(o + 16), win(o + 32), win(o + 48)

    def tile_src(table, v):
        base = pl.multiple_of((v // TR) * TR, TR)
        return table.at[pl.ds(base, TR)]

    def fire(c, buf):
        uvp, n1, n2, n3 = chunk_idx(c)
        nwins = (uvp, n1, n2, n3)
        for e in range(CH):
            dst = pl.ds((buf * CH + e) * TR, TR)
            pltpu.async_copy(tile_src(tgt_hbm, uvp[e]), u2d.at[dst],
                             sem_u[buf])
            pltpu.async_copy(tile_src(ctx_hbm, uvp[CH + e]), vp2d.at[dst],
                             sem_v[buf])
            for n in range(NNEG):
                p = 16 + e * NNEG + n
                pltpu.async_copy(
                    tile_src(ctx_hbm, nwins[p // LANES][p % LANES]),
                    vn2d.at[pl.ds((buf * CH * NNEG + e * NNEG + n) * TR, TR)],
                    sem_n[buf])

    def drain(buf):
        ur = CH * TR
        pltpu.make_async_copy(tgt_hbm.at[pl.ds(0, ur)],
                              u2d.at[pl.ds(buf * ur, ur)], sem_u[buf]).wait()
        pltpu.make_async_copy(ctx_hbm.at[pl.ds(0, ur)],
                              vp2d.at[pl.ds(buf * ur, ur)], sem_v[buf]).wait()
        nr = CH * NNEG * TR
        pltpu.make_async_copy(ctx_hbm.at[pl.ds(0, nr)],
                              vn2d.at[pl.ds(buf * nr, nr)], sem_n[buf]).wait()

    def compute(c, buf, slot):
        uvp, n1, n2, n3 = chunk_idx(c)
        nwins = (uvp, n1, n2, n3)
        for e in range(CH):
            urow = (buf * CH + e) * TR + uvp[e] % TR
            prow = (buf * CH + e) * TR + uvp[CH + e] % TR
            nrows = []
            for n in range(NNEG):
                p = 16 + e * NNEG + n
                nrows.append((buf * CH * NNEG + e * NNEG + n) * TR
                             + nwins[p // LANES][p % LANES] % TR)
            pp = jnp.zeros((LANES,), jnp.float32)
            pn = jnp.zeros((LANES,), jnp.float32)
            for j in range(D // LANES):
                sl = pl.ds(j * LANES, LANES)
                u = u2d[urow, sl]
                pp = pp + u * vp2d[prow, sl]
                vs = vn2d[nrows[0], sl]
                for n in range(1, NNEG):
                    vs = vs + vn2d[nrows[n], sl]
                pn = pn + u * vs
            part_p[slot, pl.ds(e * LANES, LANES)] = pp
            part_n[slot, pl.ds(e * LANES, LANES)] = pn

    fire(0, 0)

    def octet(q, carry):
        c0 = OCT * q
        for k in range(OCT):
            c = c0 + k
            nbuf = (k + 1) % 2
            if k < OCT - 1:
                fire(c + 1, nbuf)
            else:
                @pl.when(c + 1 < NCH)
                def _():
                    fire(c + 1, nbuf)
            drain(k % 2)
            compute(c, k % 2, k)
        orow = wid * (BW * LANES // 128) + q * OCT
        pltpu.sync_copy(part_p, pos_hbm.at[pl.ds(orow, OCT)])
        pltpu.sync_copy(part_n, neg_hbm.at[pl.ds(orow, OCT)])
        return carry

    lax.fori_loop(0, NCH // OCT, octet, 0)


def _sc_scores(tgt, ctx, idx_all, B, D, NNEG):
    BW = B // NW
    NCH = BW // CH
    mesh = plsc.VectorSubcoreMesh(core_axis_name="c", subcore_axis_name="s",
                                  num_cores=NC, num_subcores=NS)
    f32 = jnp.float32
    i32 = jnp.int32
    irows = BW * IW // CH // 128
    return pl.kernel(
        functools.partial(_sc_body, D, BW, NCH, NNEG),
        out_type=[jax.ShapeDtypeStruct((B * LANES // 128, 128), f32),
                  jax.ShapeDtypeStruct((B * LANES // 128, 128), f32)],
        mesh=mesh,
        scratch_types=[
            pltpu.VMEM((irows, 128), i32),
            pltpu.VMEM((2 * CH * TR, D), f32),
            pltpu.VMEM((2 * CH * TR, D), f32),
            pltpu.VMEM((2 * CH * NNEG * TR, D), f32),
            pltpu.VMEM((OCT, 128), f32),
            pltpu.VMEM((OCT, 128), f32),
            pltpu.SemaphoreType.DMA,
            pltpu.SemaphoreType.DMA,
            pltpu.SemaphoreType.DMA,
            pltpu.SemaphoreType.DMA,
            pltpu.SemaphoreType.DMA,
            pltpu.SemaphoreType.DMA,
        ],
    )(tgt, ctx, idx_all)


def _loss_body(p_ref, n_ref, o_ref):
    xp = p_ref[...]
    xn = n_ref[...]
    g = lax.broadcasted_iota(jnp.int32, (128, 128), 0) // LANES
    h = lax.broadcasted_iota(jnp.int32, (128, 128), 1) // LANES
    m = (g == h).astype(jnp.float32)
    dot = functools.partial(jnp.dot, precision=jax.lax.Precision.HIGHEST,
                            preferred_element_type=jnp.float32)
    ps = dot(xp, m)
    ns = dot(xn, m)
    cost = jax.nn.log_sigmoid(ps) + jax.nn.log_sigmoid(-ns)
    n_elems = xp.size // LANES
    o_ref[0, 0] = -jnp.sum(cost) / (LANES * n_elems)


def _loss_tc(pos_sc, neg_sc):
    return pl.pallas_call(
        _loss_body,
        out_shape=jax.ShapeDtypeStruct((1, 1), jnp.float32),
        in_specs=[pl.BlockSpec(memory_space=pltpu.VMEM),
                  pl.BlockSpec(memory_space=pltpu.VMEM)],
        out_specs=pl.BlockSpec(memory_space=pltpu.SMEM),
    )(pos_sc, neg_sc)


def kernel(target_table, context_table, u_pos, v_pos, v_neg):
    B = u_pos.shape[0]
    D = target_table.shape[1]
    NNEG = v_neg.shape[1]
    BW = B // NW
    i32 = jnp.int32
    nch = BW // CH
    u3 = u_pos.astype(i32).reshape(NW, nch, CH)
    v3 = v_pos.astype(i32).reshape(NW, nch, CH)
    n3 = v_neg.astype(i32).reshape(NW, nch, CH * NNEG)
    padz = jnp.zeros((NW, nch, IW - 2 * CH - CH * NNEG), i32)
    idx_all = jnp.concatenate([u3, v3, n3, padz], axis=2)
    idx_all = idx_all.reshape(NW * nch * IW // 128, 128)
    pos_sc, neg_sc = _sc_scores(target_table, context_table, idx_all,
                                B, D, NNEG)
    loss = _loss_tc(pos_sc, neg_sc)
    return loss[0, 0]
